# baseline (device time: 55622 ns/iter reference)
import jax
import jax.numpy as jnp
from jax import lax
from jax.experimental import pallas as pl
from jax.experimental.pallas import tpu as pltpu

_N_CHUNKS = 4


def kernel(x, pi):
    _, m, n = x.shape
    rows = m // _N_CHUNKS

    def body(pi_ref, x_ref, out_ref, xstage, send_buf, in_sems, send_sems,
             recv_sems):
        my_x = lax.axis_index("x")
        my_y = lax.axis_index("y")
        tgt = pi_ref[my_x]
        is_remote = tgt != my_x

        barrier_sem = pltpu.get_barrier_semaphore()

        in_copies = []
        for k in range(_N_CHUNKS):
            cp = pltpu.make_async_copy(
                x_ref.at[0, pl.ds(k * rows, rows)],
                xstage.at[k],
                in_sems.at[k],
            )
            cp.start()
            in_copies.append(cp)

        @pl.when(is_remote)
        def _():
            pl.semaphore_signal(
                barrier_sem,
                inc=1,
                device_id=(tgt, my_y),
                device_id_type=pl.DeviceIdType.MESH,
            )
            pl.semaphore_wait(barrier_sem, 1)

            rdmas = []
            for k in range(_N_CHUNKS):
                in_copies[k].wait()
                send_buf[k] = xstage[k].astype(jnp.bfloat16)
                rdma = pltpu.make_async_remote_copy(
                    src_ref=send_buf.at[k],
                    dst_ref=out_ref.at[0, pl.ds(k * rows, rows)],
                    send_sem=send_sems.at[k],
                    recv_sem=recv_sems.at[k],
                    device_id=(tgt, my_y),
                    device_id_type=pl.DeviceIdType.MESH,
                )
                rdma.start()
                rdmas.append(rdma)
            for k in range(_N_CHUNKS):
                rdmas[k].wait()

        @pl.when(jnp.logical_not(is_remote))
        def _():
            out_copies = []
            for k in range(_N_CHUNKS):
                in_copies[k].wait()
                send_buf[k] = xstage[k].astype(jnp.bfloat16)
                cp = pltpu.make_async_copy(
                    send_buf.at[k],
                    out_ref.at[0, pl.ds(k * rows, rows)],
                    send_sems.at[k],
                )
                cp.start()
                out_copies.append(cp)
            for k in range(_N_CHUNKS):
                out_copies[k].wait()

    return pl.pallas_call(
        body,
        out_shape=jax.ShapeDtypeStruct(x.shape, jnp.bfloat16),
        in_specs=[
            pl.BlockSpec(memory_space=pltpu.SMEM),
            pl.BlockSpec(memory_space=pl.ANY),
        ],
        out_specs=pl.BlockSpec(memory_space=pl.ANY),
        scratch_shapes=[
            pltpu.VMEM((_N_CHUNKS, rows, n), jnp.float32),
            pltpu.VMEM((_N_CHUNKS, rows, n), jnp.bfloat16),
            pltpu.SemaphoreType.DMA((_N_CHUNKS,)),
            pltpu.SemaphoreType.DMA((_N_CHUNKS,)),
            pltpu.SemaphoreType.DMA((_N_CHUNKS,)),
        ],
        compiler_params=pltpu.CompilerParams(collective_id=0),
    )(pi, x)


# device time: 53254 ns/iter; 1.0445x vs baseline; 1.0445x over previous
import jax
import jax.numpy as jnp
from jax import lax
from jax.experimental import pallas as pl
from jax.experimental.pallas import tpu as pltpu

_N_CHUNKS = 4


def kernel(x, pi):
    _, m, n = x.shape
    rows = m // _N_CHUNKS

    def body(pi_ref, x_ref, out_ref, xstage, send_buf, in_sems, send_sems,
             recv_sems):
        my_x = lax.axis_index("x")
        my_y = lax.axis_index("y")
        tgt = pi_ref[my_x]
        is_remote = tgt != my_x

        barrier_sem = pltpu.get_barrier_semaphore()

        in_copies = []
        for k in range(_N_CHUNKS):
            cp = pltpu.make_async_copy(
                x_ref.at[0, pl.ds(k * rows, rows)],
                xstage.at[k],
                in_sems.at[k],
            )
            cp.start()
            in_copies.append(cp)

        @pl.when(is_remote)
        def _():
            pl.semaphore_signal(
                barrier_sem,
                inc=1,
                device_id=(tgt, my_y),
                device_id_type=pl.DeviceIdType.MESH,
            )
            pl.semaphore_wait(barrier_sem, 1)

            rdmas = []
            for k in range(_N_CHUNKS):
                in_copies[k].wait()
                send_buf[k] = xstage[k].astype(jnp.bfloat16)
                rdma = pltpu.make_async_remote_copy(
                    src_ref=send_buf.at[k],
                    dst_ref=out_ref.at[0, pl.ds(k * rows, rows)],
                    send_sem=send_sems.at[k],
                    recv_sem=recv_sems.at[k],
                    device_id=(tgt, my_y),
                    device_id_type=pl.DeviceIdType.MESH,
                )
                rdma.start()
                rdmas.append(rdma)
            for k in range(_N_CHUNKS):
                rdmas[k].wait()

        @pl.when(jnp.logical_not(is_remote))
        def _():
            out_copies = []
            for k in range(_N_CHUNKS):
                in_copies[k].wait()
                send_buf[k] = xstage[k].astype(jnp.bfloat16)
                cp = pltpu.make_async_copy(
                    send_buf.at[k],
                    out_ref.at[0, pl.ds(k * rows, rows)],
                    send_sems.at[k],
                )
                cp.start()
                out_copies.append(cp)
            for k in range(_N_CHUNKS):
                out_copies[k].wait()

    return pl.pallas_call(
        body,
        out_shape=jax.ShapeDtypeStruct(x.shape, jnp.bfloat16),
        in_specs=[
            pl.BlockSpec(memory_space=pltpu.SMEM),
            pl.BlockSpec(memory_space=pltpu.MemorySpace.HBM),
        ],
        out_specs=pl.BlockSpec(memory_space=pltpu.MemorySpace.HBM),
        scratch_shapes=[
            pltpu.VMEM((_N_CHUNKS, rows, n), jnp.float32),
            pltpu.VMEM((_N_CHUNKS, rows, n), jnp.bfloat16),
            pltpu.SemaphoreType.DMA((_N_CHUNKS,)),
            pltpu.SemaphoreType.DMA((_N_CHUNKS,)),
            pltpu.SemaphoreType.DMA((_N_CHUNKS,)),
        ],
        compiler_params=pltpu.CompilerParams(collective_id=0),
    )(pi, pltpu.with_memory_space_constraint(x, pltpu.MemorySpace.HBM))


# device time: 52979 ns/iter; 1.0499x vs baseline; 1.0052x over previous
import jax
import jax.numpy as jnp
from jax import lax
from jax.experimental import pallas as pl
from jax.experimental.pallas import tpu as pltpu

_N_CHUNKS = 8


def kernel(x, pi):
    _, m, n = x.shape
    rows = m // _N_CHUNKS

    def body(pi_ref, x_ref, out_ref, xstage, send_buf, in_sems, send_sems,
             recv_sems):
        my_x = lax.axis_index("x")
        my_y = lax.axis_index("y")
        tgt = pi_ref[my_x]
        is_remote = tgt != my_x

        barrier_sem = pltpu.get_barrier_semaphore()

        in_copies = []
        for k in range(_N_CHUNKS):
            cp = pltpu.make_async_copy(
                x_ref.at[0, pl.ds(k * rows, rows)],
                xstage.at[k],
                in_sems.at[k],
            )
            cp.start()
            in_copies.append(cp)

        @pl.when(is_remote)
        def _():
            pl.semaphore_signal(
                barrier_sem,
                inc=1,
                device_id=(tgt, my_y),
                device_id_type=pl.DeviceIdType.MESH,
            )
            pl.semaphore_wait(barrier_sem, 1)

            rdmas = []
            for k in range(_N_CHUNKS):
                in_copies[k].wait()
                send_buf[k] = xstage[k].astype(jnp.bfloat16)
                rdma = pltpu.make_async_remote_copy(
                    src_ref=send_buf.at[k],
                    dst_ref=out_ref.at[0, pl.ds(k * rows, rows)],
                    send_sem=send_sems.at[k],
                    recv_sem=recv_sems.at[k],
                    device_id=(tgt, my_y),
                    device_id_type=pl.DeviceIdType.MESH,
                )
                rdma.start()
                rdmas.append(rdma)
            for k in range(_N_CHUNKS):
                rdmas[k].wait()

        @pl.when(jnp.logical_not(is_remote))
        def _():
            out_copies = []
            for k in range(_N_CHUNKS):
                in_copies[k].wait()
                send_buf[k] = xstage[k].astype(jnp.bfloat16)
                cp = pltpu.make_async_copy(
                    send_buf.at[k],
                    out_ref.at[0, pl.ds(k * rows, rows)],
                    send_sems.at[k],
                )
                cp.start()
                out_copies.append(cp)
            for k in range(_N_CHUNKS):
                out_copies[k].wait()

    return pl.pallas_call(
        body,
        out_shape=jax.ShapeDtypeStruct(x.shape, jnp.bfloat16),
        in_specs=[
            pl.BlockSpec(memory_space=pltpu.SMEM),
            pl.BlockSpec(memory_space=pltpu.MemorySpace.HBM),
        ],
        out_specs=pl.BlockSpec(memory_space=pltpu.MemorySpace.HBM),
        scratch_shapes=[
            pltpu.VMEM((_N_CHUNKS, rows, n), jnp.float32),
            pltpu.VMEM((_N_CHUNKS, rows, n), jnp.bfloat16),
            pltpu.SemaphoreType.DMA((_N_CHUNKS,)),
            pltpu.SemaphoreType.DMA((_N_CHUNKS,)),
            pltpu.SemaphoreType.DMA((_N_CHUNKS,)),
        ],
        compiler_params=pltpu.CompilerParams(collective_id=0),
    )(pi, pltpu.with_memory_space_constraint(x, pltpu.MemorySpace.HBM))
